# baseline (speedup 1.0000x reference)
import jax
import jax.numpy as jnp
from jax import lax
from jax.experimental import pallas as pl
from jax.experimental.pallas import tpu as pltpu
from jax.experimental.pallas import tpu_sc as plsc

_N = 10000
_E = 320000
_D = 128
_NP = 10240
_NCORES = 2
_NSUB = 16
_NW = _NCORES * _NSUB
_EPT = _E // _NW
_CH = 16
_M = 636
_EPTP = _M * _CH
_ROWS_PER_SUB = _NP // _NSUB


_CP = pltpu.CompilerParams(needs_layout_passes=False)


def _mesh():
    return plsc.VectorSubcoreMesh(core_axis_name="c", subcore_axis_name="s",
                                  num_cores=_NCORES, num_subcores=_NSUB)


def _sc_pre_body(srcf, dstf, ewf, zeros_hbm, i80_hbm, outd, outc,
                 srcb, dstb, ewb, degl, cntl, i80, shd, shc):
    c = lax.axis_index("c")
    s = lax.axis_index("s")
    w = c * _NSUB + s

    pltpu.sync_copy(srcf.at[w], srcb)
    pltpu.sync_copy(dstf.at[w], dstb)
    pltpu.sync_copy(ewf.at[w], ewb)
    pltpu.sync_copy(zeros_hbm.at[pl.ds(0, 80)], degl)
    pltpu.sync_copy(zeros_hbm.at[pl.ds(0, 80)], cntl)
    pltpu.sync_copy(i80_hbm, i80)

    @pl.when(s == 0)
    def _():
        pltpu.sync_copy(zeros_hbm.at[pl.ds(0, 80)], shd)
        pltpu.sync_copy(zeros_hbm.at[pl.ds(0, 80)], shc)

    ones16 = jnp.full((16,), 1.0, jnp.float32)

    @pl.loop(0, _M)
    def _(m):
        sl = pl.ds(m * _CH, _CH)
        dv = dstb[sl]
        sv = srcb[sl]
        wv = ewb[sl]
        plsc.addupdate_scatter(
            degl,
            [lax.shift_right_logical(dv, 7), lax.bitwise_and(dv, 127)],
            wv)
        plsc.addupdate_scatter(
            cntl,
            [lax.shift_right_logical(sv, 7), lax.bitwise_and(sv, 127)],
            ones16)

    plsc.subcore_barrier()
    pltpu.sync_copy(degl, shd.at[i80], add=True)
    pltpu.sync_copy(cntl, shc.at[i80], add=True)
    plsc.subcore_barrier()

    @pl.when(s == 0)
    def _():
        pltpu.sync_copy(shd, outd.at[c])
        pltpu.sync_copy(shc, outc.at[c])


def _sc_pre(srcf, dstf, ewf, zeros_np, i80):
    f = pl.kernel(
        _sc_pre_body,
        out_type=(
            jax.ShapeDtypeStruct((_NCORES, 80, 128), jnp.float32),
            jax.ShapeDtypeStruct((_NCORES, 80, 128), jnp.float32),
        ),
        mesh=_mesh(),
        compiler_params=_CP,
        scratch_types=[
            pltpu.VMEM((_EPTP,), jnp.int32),
            pltpu.VMEM((_EPTP,), jnp.int32),
            pltpu.VMEM((_EPTP,), jnp.float32),
            pltpu.VMEM((80, 128), jnp.float32),
            pltpu.VMEM((80, 128), jnp.float32),
            pltpu.VMEM((80,), jnp.int32),
            pltpu.VMEM_SHARED((80, 128), jnp.float32),
            pltpu.VMEM_SHARED((80, 128), jnp.float32),
        ],
    )
    return f(srcf, dstf, ewf, zeros_np, i80)


_G = 4
_NGRP = 3
_CB = _G * _CH


def _sc_agg_body(hs_hbm, pckf, ewf, zeros_hbm, out_hbm,
                 pckb, ewb, rows, acc, *sems):
    c = lax.axis_index("c")
    s = lax.axis_index("s")
    w = c * _NSUB + s
    gsems = sems[:_NGRP]
    ssems = sems[_NGRP:]

    rs = pl.ds(s * _ROWS_PER_SUB, _ROWS_PER_SUB)
    pltpu.sync_copy(zeros_hbm.at[rs], acc.at[rs])
    pltpu.sync_copy(pckf.at[w], pckb)
    pltpu.sync_copy(ewf.at[w], ewb)
    plsc.subcore_barrier()

    def _fire(base, g):
        for u in range(_G):
            b = g * _G + u
            iv = lax.bitwise_and(pckb[pl.ds(base + u * _CH, _CH)], 0xFFFF)
            pltpu.async_copy(hs_hbm.at[iv], rows.at[b], gsems[g])

    def _drain(base, g):
        for u in range(_G):
            iv = lax.bitwise_and(pckb[pl.ds(base + u * _CH, _CH)], 0xFFFF)
            pltpu.make_async_copy(
                hs_hbm.at[iv], rows.at[g * _G + u], gsems[g]).wait()
        scs = []
        for u in range(_G):
            b = g * _G + u
            ev = ewb[pl.ds(base + u * _CH, _CH)]
            for j in range(_CH):
                sv = jnp.broadcast_to(ev[j], (16,))
                for cc in range(_D // 16):
                    cs = pl.ds(cc * 16, 16)
                    rows[b, j, cs] = rows[b, j, cs] * sv
            dv = lax.shift_right_logical(
                pckb[pl.ds(base + u * _CH, _CH)], 16)
            scs.append(pltpu.async_copy(rows.at[b], acc.at[dv], ssems[g],
                                        add=True))
        return scs

    nit = _M // (_NGRP * _G)
    for g in range(_NGRP):
        _fire(g * _CB, g)

    @pl.loop(0, nit)
    def _(m):
        base = m * _NGRP * _CB
        s0 = _drain(base, 0)
        s1 = _drain(base + _CB, 1)
        for cp in s0:
            cp.wait()

        @pl.when(m < nit - 1)
        def _():
            _fire(base + 3 * _CB, 0)

        s2 = _drain(base + 2 * _CB, 2)
        for cp in s1:
            cp.wait()

        @pl.when(m < nit - 1)
        def _():
            _fire(base + 4 * _CB, 1)

        for cp in s2:
            cp.wait()

        @pl.when(m < nit - 1)
        def _():
            _fire(base + 5 * _CB, 2)

    plsc.subcore_barrier()
    pltpu.sync_copy(acc.at[rs], out_hbm.at[c, rs])


def _sc_agg(hs, pckf, ewf, zeros_np):
    f = pl.kernel(
        _sc_agg_body,
        out_type=jax.ShapeDtypeStruct((_NCORES, _NP, _D), jnp.float32),
        mesh=_mesh(),
        compiler_params=_CP,
        scratch_types=[
            pltpu.VMEM((_EPTP,), jnp.int32),
            pltpu.VMEM((_EPTP,), jnp.float32),
            pltpu.VMEM((_NGRP * _G, _CH, _D), jnp.float32),
            pltpu.VMEM_SHARED((_NP, _D), jnp.float32),
        ] + [pltpu.SemaphoreType.DMA] * (2 * _NGRP),
    )
    return f(hs, pckf, ewf, zeros_np)


def _tc1_body(x_ref, w0t_ref, dp0_ref, dp1_ref, hs0_ref, dinv_ref):
    deg = dp0_ref[...] + dp1_ref[...] + 1.0
    dinv = lax.rsqrt(deg)
    dinv2d = jnp.broadcast_to(dinv, (_NP, _D))
    h0 = jnp.dot(x_ref[...], w0t_ref[...],
                 preferred_element_type=jnp.float32,
                 precision=lax.Precision.HIGHEST)
    hs0_ref[...] = h0 * dinv2d
    dinv_ref[...] = dinv2d


def _tc_mid_body(pa_ref, pb_ref, hs_ref, dinv_ref, wt_ref, b_ref,
                 hs_out_ref):
    act = jax.nn.relu(dinv_ref[...] * (pa_ref[...] + pb_ref[...] + hs_ref[...])
                      + b_ref[...])
    h = jnp.dot(act, wt_ref[...], preferred_element_type=jnp.float32,
                precision=lax.Precision.HIGHEST)
    hs_out_ref[...] = h * dinv_ref[...]


def _tc_fin_body(pa_ref, pb_ref, hs_ref, dinv_ref, b2_ref,
                 w1t_ref, w1b_ref, w2t_ref, w2b_ref, w3t_ref, w3b_ref,
                 q_ref, c0_ref, c1_ref, out_ref):
    h = dinv_ref[...] * (pa_ref[...] + pb_ref[...] + hs_ref[...]) + b2_ref[...]
    mask_col = lax.broadcasted_iota(jnp.int32, (_NP, 1), 0) < _N
    hm = jnp.where(mask_col, h, 0.0)
    mean = jnp.sum(hm, axis=0, keepdims=True) * (1.0 / _N)
    alpha = jnp.tanh(
        jnp.dot(h, w1t_ref[...], preferred_element_type=jnp.float32,
                precision=lax.Precision.HIGHEST) + w1b_ref[...]
        + jnp.dot(mean, w2t_ref[...], preferred_element_type=jnp.float32,
                  precision=lax.Precision.HIGHEST) + w2b_ref[...])
    scores = jnp.sum(alpha * q_ref[...], axis=1, keepdims=True)
    scores = jnp.where(mask_col, scores, -1e30)
    smax = jnp.max(scores)
    ex = jnp.where(mask_col, jnp.exp(scores - smax), 0.0)
    attn = ex * (1.0 / jnp.sum(ex))
    ng = jnp.sum(attn * h, axis=0, keepdims=True)

    cnt = c0_ref[...] + c1_ref[...]
    iota_row = lax.broadcasted_iota(jnp.int32, (1, _NP), 1)
    cntm = jnp.where(iota_row < _N, cnt, -1.0)
    cmax = jnp.max(cntm)
    key = jnp.min(jnp.where(cntm == cmax, iota_row, _NP))
    iota_col = lax.broadcasted_iota(jnp.int32, (_NP, 1), 0)
    oh = jnp.where(iota_col == key, 1.0, 0.0)
    key_vec = jnp.sum(oh * h, axis=0, keepdims=True)

    comb = jnp.concatenate([ng, key_vec], axis=1)
    out_ref[...] = jnp.tanh(
        jnp.dot(comb, w3t_ref[...], preferred_element_type=jnp.float32,
                precision=lax.Precision.HIGHEST) + w3b_ref[...])


def _tc1(x, w0t, dp0, dp1):
    return pl.pallas_call(
        _tc1_body,
        out_shape=(jax.ShapeDtypeStruct((_NP, _D), jnp.float32),
                   jax.ShapeDtypeStruct((_NP, _D), jnp.float32)),
    )(x, w0t, dp0, dp1)


def _tc_mid(pa, pb, hs, dinv, wt, b):
    return pl.pallas_call(
        _tc_mid_body,
        out_shape=jax.ShapeDtypeStruct((_NP, _D), jnp.float32),
    )(pa, pb, hs, dinv, wt, b)


def _tc_fin(pa, pb, hs, dinv, b2, w1t, w1b, w2t, w2b, w3t, w3b, q, c0, c1):
    return pl.pallas_call(
        _tc_fin_body,
        out_shape=jax.ShapeDtypeStruct((1, _D), jnp.float32),
    )(pa, pb, hs, dinv, b2, w1t, w1b, w2t, w2b, w3t, w3b, q, c0, c1)


def kernel(x, edge_index, edge_attr, Wg0, bg0, Wg1, bg1, Wg2, bg2, q,
           W1w, W1b, W2w, W2b, W3w, W3b):
    f32 = jnp.float32
    src = edge_index[0]
    dst = edge_index[1]

    pad = _EPTP - _EPT
    srcf = jnp.pad(src.reshape(_NW, _EPT), ((0, 0), (0, pad)),
                   constant_values=_NP - 1)
    dstf = jnp.pad(dst.reshape(_NW, _EPT), ((0, 0), (0, pad)),
                   constant_values=_NP - 1)
    ewf = jnp.pad(edge_attr.reshape(_NW, _EPT), ((0, 0), (0, pad)))
    pckf = jnp.bitwise_or(jnp.left_shift(dstf, 16), srcf)

    zeros_np = jnp.zeros((_NP, _D), f32)
    i80 = jnp.arange(80, dtype=jnp.int32)

    xp = jnp.pad(x, ((0, _NP - _N), (0, 0)))

    deg_p, cnt_p = _sc_pre(srcf, dstf, ewf, zeros_np, i80)
    dp = deg_p.reshape(_NCORES, _NP, 1)
    hs0, dinv2d = _tc1(xp, Wg0.T, dp[0], dp[1])

    p0 = _sc_agg(hs0, pckf, ewf, zeros_np)
    hs1 = _tc_mid(p0[0], p0[1], hs0, dinv2d, Wg1.T, bg0.reshape(1, _D))
    p1 = _sc_agg(hs1, pckf, ewf, zeros_np)
    hs2 = _tc_mid(p1[0], p1[1], hs1, dinv2d, Wg2.T, bg1.reshape(1, _D))
    p2 = _sc_agg(hs2, pckf, ewf, zeros_np)

    cp = cnt_p.reshape(_NCORES, 1, _NP)
    nr = _tc_fin(p2[0], p2[1], hs2, dinv2d, bg2.reshape(1, _D),
                 W1w.T, W1b.reshape(1, _D), W2w.T, W2b.reshape(1, _D),
                 W3w.T, W3b.reshape(1, _D), q.reshape(1, _D),
                 cp[0], cp[1])
    return nr.reshape(_D)


# speedup vs baseline: 1.2423x; 1.2423x over previous
import jax
import jax.numpy as jnp
from jax import lax
from jax.experimental import pallas as pl
from jax.experimental.pallas import tpu as pltpu
from jax.experimental.pallas import tpu_sc as plsc

_N = 10000
_E = 320000
_D = 128
_NP = 10240
_NCORES = 2
_NSUB = 16
_NW = _NCORES * _NSUB
_EPT = _E // _NW
_CH = 16
_M = 632
_EPTP = _M * _CH
_ROWS_PER_SUB = _NP // _NSUB


_CP = pltpu.CompilerParams(needs_layout_passes=False)


def _mesh():
    return plsc.VectorSubcoreMesh(core_axis_name="c", subcore_axis_name="s",
                                  num_cores=_NCORES, num_subcores=_NSUB)


def _sc_pre_body(srcf, dstf, ewf, zeros_hbm, i80_hbm, outd, outc,
                 srcb, dstb, ewb, degl, cntl, i80, shd, shc):
    c = lax.axis_index("c")
    s = lax.axis_index("s")
    w = c * _NSUB + s

    pltpu.sync_copy(srcf.at[w], srcb)
    pltpu.sync_copy(dstf.at[w], dstb)
    pltpu.sync_copy(ewf.at[w], ewb)
    pltpu.sync_copy(zeros_hbm.at[pl.ds(0, 80)], degl)
    pltpu.sync_copy(zeros_hbm.at[pl.ds(0, 80)], cntl)
    pltpu.sync_copy(i80_hbm, i80)

    @pl.when(s == 0)
    def _():
        pltpu.sync_copy(zeros_hbm.at[pl.ds(0, 80)], shd)
        pltpu.sync_copy(zeros_hbm.at[pl.ds(0, 80)], shc)

    ones16 = jnp.full((16,), 1.0, jnp.float32)

    @pl.loop(0, _M)
    def _(m):
        sl = pl.ds(m * _CH, _CH)
        dv = dstb[sl]
        sv = srcb[sl]
        wv = ewb[sl]
        plsc.addupdate_scatter(
            degl,
            [lax.shift_right_logical(dv, 7), lax.bitwise_and(dv, 127)],
            wv)
        plsc.addupdate_scatter(
            cntl,
            [lax.shift_right_logical(sv, 7), lax.bitwise_and(sv, 127)],
            ones16)

    plsc.subcore_barrier()
    pltpu.sync_copy(degl, shd.at[i80], add=True)
    pltpu.sync_copy(cntl, shc.at[i80], add=True)
    plsc.subcore_barrier()

    @pl.when(s == 0)
    def _():
        pltpu.sync_copy(shd, outd.at[c])
        pltpu.sync_copy(shc, outc.at[c])


def _sc_pre(srcf, dstf, ewf, zeros_np, i80):
    f = pl.kernel(
        _sc_pre_body,
        out_type=(
            jax.ShapeDtypeStruct((_NCORES, 80, 128), jnp.float32),
            jax.ShapeDtypeStruct((_NCORES, 80, 128), jnp.float32),
        ),
        mesh=_mesh(),
        compiler_params=_CP,
        scratch_types=[
            pltpu.VMEM((_EPTP,), jnp.int32),
            pltpu.VMEM((_EPTP,), jnp.int32),
            pltpu.VMEM((_EPTP,), jnp.float32),
            pltpu.VMEM((80, 128), jnp.float32),
            pltpu.VMEM((80, 128), jnp.float32),
            pltpu.VMEM((80,), jnp.int32),
            pltpu.VMEM_SHARED((80, 128), jnp.float32),
            pltpu.VMEM_SHARED((80, 128), jnp.float32),
        ],
    )
    return f(srcf, dstf, ewf, zeros_np, i80)


_UNROLL = 4


def _sc_agg_body(hs_hbm, srcf, dstf, ewf, zeros_hbm, out_hbm,
                 srcb, dstb, ewb, rows, acc, *sems):
    c = lax.axis_index("c")
    s = lax.axis_index("s")
    w = c * _NSUB + s

    rs = pl.ds(s * _ROWS_PER_SUB, _ROWS_PER_SUB)
    pltpu.sync_copy(zeros_hbm.at[rs], acc.at[rs])
    pltpu.sync_copy(srcf.at[w], srcb)
    pltpu.sync_copy(dstf.at[w], dstb)
    pltpu.sync_copy(ewf.at[w], ewb)
    plsc.subcore_barrier()

    def _fire(base, boff):
        for u in range(_UNROLL):
            iv = srcb[pl.ds(base + u * _CH, _CH)]
            pltpu.async_copy(hs_hbm.at[iv], rows.at[boff + u], sems[boff + u])

    def _drain(base, boff):
        for u in range(_UNROLL):
            pltpu.make_async_copy(
                hs_hbm.at[srcb[pl.ds(base + u * _CH, _CH)]],
                rows.at[boff + u], sems[boff + u]).wait()
            ev = ewb[pl.ds(base + u * _CH, _CH)]
            for j in range(_CH):
                sv = jnp.broadcast_to(ev[j], (16,))
                for cc in range(_D // 16):
                    cs = pl.ds(cc * 16, 16)
                    rows[boff + u, j, cs] = rows[boff + u, j, cs] * sv
            dv = dstb[pl.ds(base + u * _CH, _CH)]
            pltpu.sync_copy(rows.at[boff + u], acc.at[dv], add=True)

    grp = _UNROLL * _CH
    nit = _M // (2 * _UNROLL)
    _fire(0, 0)

    @pl.loop(0, nit)
    def _(m):
        base = m * 2 * grp
        _fire(base + grp, _UNROLL)
        _drain(base, 0)
        @pl.when(m < nit - 1)
        def _():
            _fire(base + 2 * grp, 0)
        _drain(base + grp, _UNROLL)

    plsc.subcore_barrier()
    pltpu.sync_copy(acc.at[rs], out_hbm.at[c, rs])


def _sc_agg(hs, srcf, dstf, ewf, zeros_np):
    f = pl.kernel(
        _sc_agg_body,
        out_type=jax.ShapeDtypeStruct((_NCORES, _NP, _D), jnp.float32),
        mesh=_mesh(),
        compiler_params=_CP,
        scratch_types=[
            pltpu.VMEM((_EPTP,), jnp.int32),
            pltpu.VMEM((_EPTP,), jnp.int32),
            pltpu.VMEM((_EPTP,), jnp.float32),
            pltpu.VMEM((2 * _UNROLL, _CH, _D), jnp.float32),
            pltpu.VMEM_SHARED((_NP, _D), jnp.float32),
        ] + [pltpu.SemaphoreType.DMA] * (2 * _UNROLL),
    )
    return f(hs, srcf, dstf, ewf, zeros_np)


def _tc1_body(x_ref, w0t_ref, dp0_ref, dp1_ref, hs0_ref, dinv_ref):
    deg = dp0_ref[...] + dp1_ref[...] + 1.0
    dinv = lax.rsqrt(deg)
    dinv2d = jnp.broadcast_to(dinv, (_NP, _D))
    h0 = jnp.dot(x_ref[...], w0t_ref[...],
                 preferred_element_type=jnp.float32,
                 precision=lax.Precision.HIGHEST)
    hs0_ref[...] = h0 * dinv2d
    dinv_ref[...] = dinv2d


def _tc_mid_body(pa_ref, pb_ref, hs_ref, dinv_ref, wt_ref, b_ref,
                 hs_out_ref):
    act = jax.nn.relu(dinv_ref[...] * (pa_ref[...] + pb_ref[...] + hs_ref[...])
                      + b_ref[...])
    h = jnp.dot(act, wt_ref[...], preferred_element_type=jnp.float32,
                precision=lax.Precision.HIGHEST)
    hs_out_ref[...] = h * dinv_ref[...]


def _tc_fin_body(pa_ref, pb_ref, hs_ref, dinv_ref, b2_ref,
                 w1t_ref, w1b_ref, w2t_ref, w2b_ref, w3t_ref, w3b_ref,
                 q_ref, c0_ref, c1_ref, out_ref):
    h = dinv_ref[...] * (pa_ref[...] + pb_ref[...] + hs_ref[...]) + b2_ref[...]
    mask_col = lax.broadcasted_iota(jnp.int32, (_NP, 1), 0) < _N
    hm = jnp.where(mask_col, h, 0.0)
    mean = jnp.sum(hm, axis=0, keepdims=True) * (1.0 / _N)
    alpha = jnp.tanh(
        jnp.dot(h, w1t_ref[...], preferred_element_type=jnp.float32,
                precision=lax.Precision.HIGHEST) + w1b_ref[...]
        + jnp.dot(mean, w2t_ref[...], preferred_element_type=jnp.float32,
                  precision=lax.Precision.HIGHEST) + w2b_ref[...])
    scores = jnp.sum(alpha * q_ref[...], axis=1, keepdims=True)
    scores = jnp.where(mask_col, scores, -1e30)
    smax = jnp.max(scores)
    ex = jnp.where(mask_col, jnp.exp(scores - smax), 0.0)
    attn = ex * (1.0 / jnp.sum(ex))
    ng = jnp.sum(attn * h, axis=0, keepdims=True)

    cnt = c0_ref[...] + c1_ref[...]
    iota_row = lax.broadcasted_iota(jnp.int32, (1, _NP), 1)
    cntm = jnp.where(iota_row < _N, cnt, -1.0)
    cmax = jnp.max(cntm)
    key = jnp.min(jnp.where(cntm == cmax, iota_row, _NP))
    iota_col = lax.broadcasted_iota(jnp.int32, (_NP, 1), 0)
    oh = jnp.where(iota_col == key, 1.0, 0.0)
    key_vec = jnp.sum(oh * h, axis=0, keepdims=True)

    comb = jnp.concatenate([ng, key_vec], axis=1)
    out_ref[...] = jnp.tanh(
        jnp.dot(comb, w3t_ref[...], preferred_element_type=jnp.float32,
                precision=lax.Precision.HIGHEST) + w3b_ref[...])


def _tc1(x, w0t, dp0, dp1):
    return pl.pallas_call(
        _tc1_body,
        out_shape=(jax.ShapeDtypeStruct((_NP, _D), jnp.float32),
                   jax.ShapeDtypeStruct((_NP, _D), jnp.float32)),
    )(x, w0t, dp0, dp1)


def _tc_mid(pa, pb, hs, dinv, wt, b):
    return pl.pallas_call(
        _tc_mid_body,
        out_shape=jax.ShapeDtypeStruct((_NP, _D), jnp.float32),
    )(pa, pb, hs, dinv, wt, b)


def _tc_fin(pa, pb, hs, dinv, b2, w1t, w1b, w2t, w2b, w3t, w3b, q, c0, c1):
    return pl.pallas_call(
        _tc_fin_body,
        out_shape=jax.ShapeDtypeStruct((1, _D), jnp.float32),
    )(pa, pb, hs, dinv, b2, w1t, w1b, w2t, w2b, w3t, w3b, q, c0, c1)


def kernel(x, edge_index, edge_attr, Wg0, bg0, Wg1, bg1, Wg2, bg2, q,
           W1w, W1b, W2w, W2b, W3w, W3b):
    f32 = jnp.float32
    src = edge_index[0]
    dst = edge_index[1]

    pad = _EPTP - _EPT
    srcf = jnp.pad(src.reshape(_NW, _EPT), ((0, 0), (0, pad)),
                   constant_values=_NP - 1)
    dstf = jnp.pad(dst.reshape(_NW, _EPT), ((0, 0), (0, pad)),
                   constant_values=_NP - 1)
    ewf = jnp.pad(edge_attr.reshape(_NW, _EPT), ((0, 0), (0, pad)))

    zeros_np = jnp.zeros((_NP, _D), f32)
    i80 = jnp.arange(80, dtype=jnp.int32)

    xp = jnp.pad(x, ((0, _NP - _N), (0, 0)))

    deg_p, cnt_p = _sc_pre(srcf, dstf, ewf, zeros_np, i80)
    dp = deg_p.reshape(_NCORES, _NP, 1)
    hs0, dinv2d = _tc1(xp, Wg0.T, dp[0], dp[1])

    p0 = _sc_agg(hs0, srcf, dstf, ewf, zeros_np)
    hs1 = _tc_mid(p0[0], p0[1], hs0, dinv2d, Wg1.T, bg0.reshape(1, _D))
    p1 = _sc_agg(hs1, srcf, dstf, ewf, zeros_np)
    hs2 = _tc_mid(p1[0], p1[1], hs1, dinv2d, Wg2.T, bg1.reshape(1, _D))
    p2 = _sc_agg(hs2, srcf, dstf, ewf, zeros_np)

    cp = cnt_p.reshape(_NCORES, 1, _NP)
    nr = _tc_fin(p2[0], p2[1], hs2, dinv2d, bg2.reshape(1, _D),
                 W1w.T, W1b.reshape(1, _D), W2w.T, W2b.reshape(1, _D),
                 W3w.T, W3b.reshape(1, _D), q.reshape(1, _D),
                 cp[0], cp[1])
    return nr.reshape(_D)
